# baseline (speedup 1.0000x reference)
import functools

import jax
import jax.numpy as jnp
from jax import lax
from jax.experimental import pallas as pl
from jax.experimental.pallas import tpu as pltpu
from jax.experimental.pallas import tpu_sc as plsc

NW = 32
LANE = 128
NBUF = 4


@functools.lru_cache(maxsize=None)
def _build(nb, nf, D):
    B = nb * nf
    nsub = D // 8
    tile = 8 * LANE
    assert D % 8 == 0 and nb % LANE == 0 and B % (NW * LANE) == 0
    nchunk = B // (NW * LANE)
    tpj = nb // LANE

    mesh = plsc.VectorSubcoreMesh(core_axis_name="c", subcore_axis_name="s")

    @functools.partial(
        pl.kernel,
        out_type=jax.ShapeDtypeStruct((B * D,), jnp.float32),
        mesh=mesh,
        scratch_types=[
            pltpu.VMEM((nchunk * LANE,), jnp.int32),
            pltpu.VMEM((NBUF, LANE, D), jnp.float32),
            pltpu.VMEM((NBUF, nsub * tile), jnp.float32),
            pltpu.SemaphoreType.DMA((NBUF,)),
            pltpu.SemaphoreType.DMA((NBUF,)),
        ],
        compiler_params=pltpu.CompilerParams(use_tc_tiling_on_sc=False,
                                             needs_layout_passes=False),
    )
    def gather_kernel(table, idx_hbm, out_hbm, idx_v, rows, tiles, gsem, ssem):
        w = lax.axis_index("s") * 2 + lax.axis_index("c")
        pltpu.sync_copy(idx_hbm.at[pl.ds(w * nchunk * LANE, nchunk * LANE)],
                        idx_v)
        cbase = w * nchunk
        lanes = lax.broadcasted_iota(jnp.int32, (16,), 0)
        bidxs = [lanes + 16 * k for k in range(LANE // 16)]
        fvs = [jnp.full((16,), f, jnp.int32) for f in range(D)]

        def start_gather(c_local, b):
            pltpu.async_copy(table.at[idx_v.at[pl.ds(c_local * LANE, LANE)]],
                             rows.at[b], gsem.at[b])

        def wait_gather(b):
            pltpu.make_async_copy(table.at[idx_v.at[pl.ds(0, LANE)]],
                                  rows.at[b], gsem.at[b]).wait()

        def transpose(b):
            rr = rows.at[b]
            tt = tiles.at[b]
            for f in range(D):
                for k in range(LANE // 16):
                    v = plsc.load_gather(rr, [bidxs[k], fvs[f]])
                    tt[pl.ds(f * LANE + 16 * k, 16)] = v

        def start_scatter(j, t, b):
            for s in range(nsub):
                off = ((j * nsub + s) * tpj + t) * tile
                pltpu.async_copy(tiles.at[b, pl.ds(s * tile, tile)],
                                 out_hbm.at[pl.ds(off, tile)], ssem.at[b])

        def wait_scatter(b):
            pltpu.make_async_copy(out_hbm.at[pl.ds(0, nsub * tile)],
                                  tiles.at[b], ssem.at[b]).wait()

        for b in range(min(NBUF - 1, nchunk)):
            start_gather(b, b)

        @pl.loop(0, nchunk)
        def _(c_local):
            b = lax.rem(c_local, NBUF)
            c = cbase + c_local
            j = c // tpj
            t = lax.rem(c, tpj)
            wait_gather(b)

            @pl.when(c_local >= NBUF)
            def _():
                wait_scatter(b)

            transpose(b)
            start_scatter(j, t, b)

            @pl.when(c_local + (NBUF - 1) < nchunk)
            def _():
                start_gather(c_local + (NBUF - 1),
                             lax.rem(c_local + (NBUF - 1), NBUF))

        for b in range(min(NBUF, nchunk)):
            wait_scatter(b)

    return gather_kernel


def kernel(image, gather_indices):
    nb, nf, _ = gather_indices.shape
    B = nb * nf
    D = image.shape[1]
    idx = jnp.transpose(gather_indices, (1, 2, 0)).reshape(B).astype(jnp.int32)
    outb = _build(nb, nf, D)(image, idx)
    out5 = outb.reshape(nf, D // 8, nb // LANE, 8, LANE)
    return jnp.transpose(out5, (2, 4, 0, 1, 3)).reshape(nb, nf, D)


# speedup vs baseline: 1.0989x; 1.0989x over previous
import functools

import jax
import jax.numpy as jnp
from jax import lax
from jax.experimental import pallas as pl
from jax.experimental.pallas import tpu as pltpu
from jax.experimental.pallas import tpu_sc as plsc

NW = 32
LANE = 128
NBUF = 4


@functools.lru_cache(maxsize=None)
def _build(nb, nf, D):
    B = nb * nf
    nsub = D // 8
    tile = 8 * LANE
    assert D % 8 == 0 and nb % LANE == 0 and B % (NW * LANE) == 0
    nchunk = B // (NW * LANE)
    tpj = nb // LANE

    mesh = plsc.VectorSubcoreMesh(core_axis_name="c", subcore_axis_name="s")

    @functools.partial(
        pl.kernel,
        out_type=jax.ShapeDtypeStruct((B * D,), jnp.float32),
        mesh=mesh,
        scratch_types=[
            pltpu.VMEM((nchunk * LANE,), jnp.int32),
            pltpu.VMEM((NBUF, LANE, D), jnp.float32),
            pltpu.VMEM((NBUF, nsub * tile), jnp.float32),
            pltpu.SemaphoreType.DMA((NBUF,)),
            pltpu.SemaphoreType.DMA((NBUF,)),
        ],
        compiler_params=pltpu.CompilerParams(use_tc_tiling_on_sc=False,
                                             needs_layout_passes=False),
    )
    def gather_kernel(table, idx_hbm, out_hbm, idx_v, rows, tiles, gsem, ssem):
        w = lax.axis_index("s") * 2 + lax.axis_index("c")
        pltpu.sync_copy(idx_hbm.at[pl.ds(w * nchunk * LANE, nchunk * LANE)],
                        idx_v)
        cbase = w * nchunk
        lanes = lax.broadcasted_iota(jnp.int32, (16,), 0)
        bidxs = [lanes + 16 * k for k in range(LANE // 16)]
        fvs = [jnp.full((16,), f, jnp.int32) for f in range(D)]

        def start_gather(c_local, b):
            pltpu.async_copy(table.at[idx_v.at[pl.ds(c_local * LANE, LANE)]],
                             rows.at[b], gsem.at[b])

        def wait_gather(b):
            pltpu.make_async_copy(table.at[idx_v.at[pl.ds(0, LANE)]],
                                  rows.at[b], gsem.at[b]).wait()

        def transpose(b):
            rr = rows.at[b]
            tt = tiles.at[b]
            for f0 in range(0, D, 2):
                vs = [plsc.load_gather(rr, [bidxs[k % 8], fvs[f0 + (k // 8)]])
                      for k in range(16)]
                for k in range(16):
                    f = f0 + (k // 8)
                    tt[pl.ds(f * LANE + 16 * (k % 8), 16)] = vs[k]

        def start_scatter(j, t, b):
            for s in range(nsub):
                off = ((j * nsub + s) * tpj + t) * tile
                pltpu.async_copy(tiles.at[b, pl.ds(s * tile, tile)],
                                 out_hbm.at[pl.ds(off, tile)], ssem.at[b])

        def wait_scatter(b):
            pltpu.make_async_copy(out_hbm.at[pl.ds(0, nsub * tile)],
                                  tiles.at[b], ssem.at[b]).wait()

        for b in range(min(NBUF - 1, nchunk)):
            start_gather(b, b)

        @pl.loop(0, nchunk)
        def _(c_local):
            b = lax.rem(c_local, NBUF)
            c = cbase + c_local
            j = c // tpj
            t = lax.rem(c, tpj)
            wait_gather(b)

            @pl.when(c_local >= NBUF)
            def _():
                wait_scatter(b)

            transpose(b)
            start_scatter(j, t, b)

            @pl.when(c_local + (NBUF - 1) < nchunk)
            def _():
                start_gather(c_local + (NBUF - 1),
                             lax.rem(c_local + (NBUF - 1), NBUF))

        for b in range(min(NBUF, nchunk)):
            wait_scatter(b)

    return gather_kernel


def kernel(image, gather_indices):
    nb, nf, _ = gather_indices.shape
    B = nb * nf
    D = image.shape[1]
    idx = jnp.transpose(gather_indices, (1, 2, 0)).reshape(B).astype(jnp.int32)
    outb = _build(nb, nf, D)(image, idx)
    out5 = outb.reshape(nf, D // 8, nb // LANE, 8, LANE)
    return jnp.transpose(out5, (2, 4, 0, 1, 3)).reshape(nb, nf, D)
